# baseline (speedup 1.0000x reference)
import jax
import jax.numpy as jnp
from jax import lax
from jax.experimental import pallas as pl
from jax.experimental.pallas import tpu as pltpu
from jax.experimental.pallas import tpu_sc as plsc

H = 128
WD = 128
ROWS = H + 2
COLS = 144
L = 16
NCH = WD // L


def _sc_body(xp_hbm, op_hbm, out_hbm, w_v, op_v, comb_v, ktmp_v, stmp_v):
    c = lax.axis_index("c")
    s = lax.axis_index("s")

    @pl.when(jnp.logical_and(c == 0, s == 0))
    def _():
        pltpu.sync_copy(xp_hbm, w_v)
        pltpu.sync_copy(op_hbm, op_v)
        zero_i = jnp.zeros((L,), jnp.int32)
        one_i = jnp.ones((L,), jnp.int32)
        neg1_i = jnp.full((L,), -1, jnp.int32)
        one_f = jnp.ones((L,), jnp.float32)
        iota = lax.iota(jnp.int32, L)
        comb_v[pl.ds(0, L)] = zero_i
        pos_q = [iota + jnp.full((L,), q * L, jnp.int32) for q in range(NCH)]
        lane_q = [jnp.full((L,), q, jnp.int32) for q in range(NCH)]
        tot_idx = jnp.minimum(iota, jnp.full((L,), NCH - 1, jnp.int32)) * L \
            + jnp.full((L,), L - 1, jnp.int32)
        prev_lane = jnp.maximum(iota - one_i, zero_i)
        is_lane0 = iota == zero_i

        def row_body(row, carry_unused):
            rb = row * COLS
            kloc = [None] * NCH
            csum = [None] * NCH
            t0s = [None] * NCH
            for q in range(NCH):
                o = q * L
                N = w_v[pl.ds(rb - COLS + o + 1, L)]
                NW = w_v[pl.ds(rb - COLS + o, L)]
                NE = w_v[pl.ds(rb - COLS + o + 2, L)]
                b = w_v[pl.ds(rb + o + 1, L)]
                E = w_v[pl.ds(rb + o + 2, L)]
                S = w_v[pl.ds(rb + COLS + o + 1, L)]
                SW = w_v[pl.ds(rb + COLS + o, L)]
                SE = w_v[pl.ds(rb + COLS + o + 2, L)]

                K = one_f - (N + E + S) + NE * N * E + SE * S * E
                M = NW * N + SW * S - one_f
                sgn = one_f - b - b
                a0 = jnp.where(K == sgn, one_i, zero_i)
                a1 = jnp.where(K + M == sgn, one_i, zero_i)
                bi = b.astype(jnp.int32)
                t0 = plsc.load_gather(op_v, [a0 + a0 + bi])
                t1 = plsc.load_gather(op_v, [a1 + a1 + bi])

                is_const = t0 == t1
                d = jnp.where(is_const, zero_i, t0)
                kq = plsc.cummax(jnp.where(is_const, pos_q[q], neg1_i))
                sq = plsc.cumsum(d)
                ktmp_v[pl.ds(o, L)] = kq
                stmp_v[pl.ds(o, L)] = sq
                kloc[q], csum[q], t0s[q] = kq, sq, t0

            tmax = plsc.load_gather(ktmp_v, [tot_idx])
            tsum = plsc.load_gather(stmp_v, [tot_idx])
            pmax = plsc.cummax(tmax)
            psum = plsc.cumsum(tsum)
            gmax = pmax.at[prev_lane].get(mode="promise_in_bounds")
            gsum = psum.at[prev_lane].get(mode="promise_in_bounds")
            exc_max = jnp.where(is_lane0, neg1_i, gmax)
            exc_sum = jnp.where(is_lane0, zero_i, gsum)

            for q in range(NCH):
                o = q * L
                maxc = exc_max.at[lane_q[q]].get(mode="promise_in_bounds")
                sumc = exc_sum.at[lane_q[q]].get(mode="promise_in_bounds")
                kidx = jnp.maximum(kloc[q], maxc)
                xpar = (csum[q] + sumc) & one_i
                comb_v[pl.ds(o + 1, L)] = jnp.bitwise_xor(t0s[q], xpar)
                gath = plsc.load_gather(comb_v, [kidx + one_i])
                cbits = jnp.bitwise_xor(gath, xpar)
                w_v[pl.ds(rb + o + 1, L)] = cbits.astype(jnp.float32)
            return carry_unused

        lax.fori_loop(1, H + 1, row_body, jnp.int32(0))
        pltpu.sync_copy(w_v, out_hbm)


_mesh = plsc.VectorSubcoreMesh(core_axis_name="c", subcore_axis_name="s")

_sc_call = pl.kernel(
    _sc_body,
    out_type=jax.ShapeDtypeStruct((ROWS * COLS,), jnp.float32),
    mesh=_mesh,
    scratch_types=[
        pltpu.VMEM((ROWS * COLS,), jnp.float32),
        pltpu.VMEM((L,), jnp.int32),
        pltpu.VMEM((COLS,), jnp.int32),
        pltpu.VMEM((WD,), jnp.int32),
        pltpu.VMEM((WD,), jnp.int32),
    ],
    compiler_params=pltpu.CompilerParams(needs_layout_passes=False),
)


@jax.jit
def kernel(x, operation):
    xp = jnp.zeros((ROWS, COLS), jnp.float32)
    xp = xp.at[1:H + 1, 1:WD + 1].set(x[0])
    opi = jnp.zeros((L,), jnp.int32).at[:4].set(operation.astype(jnp.int32))
    out = _sc_call(xp.reshape(ROWS * COLS), opi)
    return out.reshape(ROWS, COLS)[1:H + 1, 1:WD + 1][None]
